# baseline (speedup 1.0000x reference)
import functools

import jax
import jax.numpy as jnp
from jax import lax
from jax.experimental import pallas as pl
from jax.experimental.pallas import tpu as pltpu
from jax.experimental.pallas import tpu_sc as plsc

D = 32
NW = 32
NC = 2


def _rsqrt(t):
    i = lax.bitcast_convert_type(t, jnp.int32)
    i = jnp.int32(0x5F3759DF) - lax.shift_right_logical(i, 1)
    y = lax.bitcast_convert_type(i, jnp.float32)
    half = jnp.float32(0.5)
    three_half = jnp.float32(1.5)
    for _ in range(3):
        y = y * (three_half - half * t * y * y)
    return y


def _fused_sc(tflat, idx, batch):
    per_w = batch // NW
    slots = 2 * per_w
    groups = slots // 16
    vrows = slots // 4
    mesh = plsc.VectorSubcoreMesh(core_axis_name="c", subcore_axis_name="s")

    @functools.partial(
        pl.kernel,
        out_type=jax.ShapeDtypeStruct((batch,), jnp.float32),
        mesh=mesh,
        compiler_params=pltpu.CompilerParams(
            use_tc_tiling_on_sc=False,
            needs_layout_passes=False,
            skip_device_barrier=True,
            disable_semaphore_checks=True,
            disable_bounds_checks=True,
        ),
        scratch_types=[
            pltpu.VMEM((slots // 128, 128), jnp.int32),
            pltpu.VMEM((vrows, 128), jnp.float32),
            pltpu.VMEM((per_w,), jnp.float32),
            pltpu.SemaphoreType.DMA,
        ],
    )
    def k(tflat_hbm, idx_hbm, out_hbm, idx_v, rows_v, out_v, sem):
        wid = lax.axis_index("s") * NC + lax.axis_index("c")
        pltpu.sync_copy(idx_hbm.at[wid], idx_v)

        def fire(g, _):
            ivec = idx_v[g // 8, pl.ds((g % 8) * 16, 16)]
            for l in range(16):
                off = pl.multiple_of(ivec[l] * D, 8)
                pltpu.async_copy(
                    tflat_hbm.at[pl.ds(off, D)],
                    rows_v.at[g * 4 + l // 4, pl.ds((l % 4) * 32, 32)],
                    sem,
                )
            return 0

        lax.fori_loop(0, groups, fire, 0)

        def drain(j, _):
            pltpu.make_async_copy(
                tflat_hbm.at[pl.ds(0, D)], rows_v.at[0, pl.ds(0, 32)], sem
            ).wait()
            return 0

        lax.fori_loop(0, slots, drain, 0)

        def comp(g, _):
            lanes = lax.iota(jnp.int32, 16)
            onehots = [lanes == jnp.int32(l) for l in range(16)]
            dot = jnp.zeros((16,), jnp.float32)
            s1 = jnp.zeros((16,), jnp.float32)
            s2 = jnp.zeros((16,), jnp.float32)
            for l in range(16):
                row = g * 8 + l // 2
                col = (l % 2) * 64
                a0 = rows_v[row, pl.ds(col, 16)]
                a1 = rows_v[row, pl.ds(col + 16, 16)]
                b0 = rows_v[row, pl.ds(col + 32, 16)]
                b1 = rows_v[row, pl.ds(col + 48, 16)]
                dot = jnp.where(onehots[l], jnp.sum(a0 * b0 + a1 * b1), dot)
                s1 = jnp.where(onehots[l], jnp.sum(a0 * a0 + a1 * a1), s1)
                s2 = jnp.where(onehots[l], jnp.sum(b0 * b0 + b1 * b1), s2)
            eps2 = jnp.float32(1e-16)
            t = jnp.maximum(s1, eps2) * jnp.maximum(s2, eps2)
            sim = dot * _rsqrt(t)
            out_v[pl.ds(g * 16, 16)] = jnp.float32(0.5) + jnp.float32(0.5) * sim
            return 0

        lax.fori_loop(0, per_w // 16, comp, 0)
        base = pl.multiple_of(wid * per_w, 8)
        pltpu.sync_copy(out_v, out_hbm.at[pl.ds(base, per_w)])

    return k(tflat, idx)


def kernel(x, table):
    x = x.reshape(-1, 2)
    batch = x.shape[0]
    slots = (2 * batch) // NW
    idx = x.astype(jnp.int32).reshape(NW, slots // 128, 128)
    return _fused_sc(table.reshape(-1), idx, batch)


# speedup vs baseline: 1.0111x; 1.0111x over previous
import functools

import jax
import jax.numpy as jnp
from jax import lax
from jax.experimental import pallas as pl
from jax.experimental.pallas import tpu as pltpu
from jax.experimental.pallas import tpu_sc as plsc

D = 32
CHUNK = 128
NW = 32
NC = 2


def _rsqrt(t):
    i = lax.bitcast_convert_type(t, jnp.int32)
    i = jnp.int32(0x5F3759DF) - lax.shift_right_logical(i, 1)
    y = lax.bitcast_convert_type(i, jnp.float32)
    half = jnp.float32(0.5)
    three_half = jnp.float32(1.5)
    for _ in range(3):
        y = y * (three_half - half * t * y * y)
    return y


def _fused_sc(table, idx, batch):
    per_w = batch // NW
    slots = 2 * per_w
    n_chunks = slots // CHUNK
    mesh = plsc.VectorSubcoreMesh(core_axis_name="c", subcore_axis_name="s")

    @functools.partial(
        pl.kernel,
        out_type=jax.ShapeDtypeStruct((batch,), jnp.float32),
        mesh=mesh,
        compiler_params=pltpu.CompilerParams(
            use_tc_tiling_on_sc=False,
            needs_layout_passes=False,
            skip_device_barrier=True,
            disable_semaphore_checks=True,
            disable_bounds_checks=True,
        ),
        scratch_types=[
            pltpu.VMEM((n_chunks, CHUNK), jnp.int32),
            pltpu.VMEM((slots, D), jnp.float32),
            pltpu.VMEM((per_w,), jnp.float32),
            pltpu.SemaphoreType.DMA,
        ],
    )
    def k(table_hbm, idx_hbm, out_hbm, idx_v, rows_v, out_v, sem):
        wid = lax.axis_index("s") * NC + lax.axis_index("c")
        pltpu.sync_copy(idx_hbm.at[wid], idx_v)
        handles = []
        for c in range(n_chunks):
            handles.append(
                pltpu.async_copy(
                    table_hbm.at[idx_v.at[c]],
                    rows_v.at[pl.ds(c * CHUNK, CHUNK)],
                    sem,
                )
            )
        for h in handles:
            h.wait()

        def comp(g, _):
            lanes = lax.iota(jnp.int32, 16)
            onehots = [lanes == jnp.int32(l) for l in range(16)]
            dot = jnp.zeros((16,), jnp.float32)
            s1 = jnp.zeros((16,), jnp.float32)
            s2 = jnp.zeros((16,), jnp.float32)
            for l in range(16):
                row = g * 32 + 2 * l
                a0 = rows_v[row, pl.ds(0, 16)]
                a1 = rows_v[row, pl.ds(16, 16)]
                b0 = rows_v[row + 1, pl.ds(0, 16)]
                b1 = rows_v[row + 1, pl.ds(16, 16)]
                dot = jnp.where(onehots[l], jnp.sum(a0 * b0 + a1 * b1), dot)
                s1 = jnp.where(onehots[l], jnp.sum(a0 * a0 + a1 * a1), s1)
                s2 = jnp.where(onehots[l], jnp.sum(b0 * b0 + b1 * b1), s2)
            eps2 = jnp.float32(1e-16)
            t = jnp.maximum(s1, eps2) * jnp.maximum(s2, eps2)
            sim = dot * _rsqrt(t)
            out_v[pl.ds(g * 16, 16)] = jnp.float32(0.5) + jnp.float32(0.5) * sim
            return 0

        lax.fori_loop(0, per_w // 16, comp, 0)
        base = pl.multiple_of(wid * per_w, 8)
        pltpu.sync_copy(out_v, out_hbm.at[pl.ds(base, per_w)])

    return k(table, idx)


def kernel(x, table):
    x = x.reshape(-1, 2)
    batch = x.shape[0]
    slots = (2 * batch) // NW
    idx = x.astype(jnp.int32).reshape(NW, slots // CHUNK, CHUNK)
    return _fused_sc(table, idx, batch)


# speedup vs baseline: 1.5947x; 1.5773x over previous
import functools

import jax
import jax.numpy as jnp
from jax import lax
from jax.experimental import pallas as pl
from jax.experimental.pallas import tpu as pltpu
from jax.experimental.pallas import tpu_sc as plsc

D = 32
NW = 32
NC = 2


def _rsqrt(t):
    i = lax.bitcast_convert_type(t, jnp.int32)
    i = jnp.int32(0x5F3759DF) - lax.shift_right_logical(i, 1)
    y = lax.bitcast_convert_type(i, jnp.float32)
    half = jnp.float32(0.5)
    three_half = jnp.float32(1.5)
    for _ in range(3):
        y = y * (three_half - half * t * y * y)
    return y


def _fused_sc(table, idx, batch):
    per_w = batch // NW
    slots = 2 * per_w
    groups = slots // 16
    vrows = slots // 4
    mesh = plsc.VectorSubcoreMesh(core_axis_name="c", subcore_axis_name="s")

    @functools.partial(
        pl.kernel,
        out_type=jax.ShapeDtypeStruct((batch,), jnp.float32),
        mesh=mesh,
        compiler_params=pltpu.CompilerParams(
            use_tc_tiling_on_sc=True,
            needs_layout_passes=False,
            skip_device_barrier=True,
            disable_semaphore_checks=True,
            disable_bounds_checks=True,
        ),
        scratch_types=[
            pltpu.VMEM((slots // 128, 128), jnp.int32),
            pltpu.VMEM((vrows, 128), jnp.float32),
            pltpu.VMEM((per_w,), jnp.float32),
            pltpu.SemaphoreType.DMA,
        ],
    )
    def k(table_hbm, idx_hbm, out_hbm, idx_v, rows_v, out_v, sem):
        wid = lax.axis_index("s") * NC + lax.axis_index("c")
        pltpu.sync_copy(idx_hbm.at[wid], idx_v)

        def fire(g, _):
            ivec = idx_v[g // 8, pl.ds((g % 8) * 16, 16)]
            for l in range(16):
                pltpu.async_copy(
                    table_hbm.at[ivec[l]],
                    rows_v.at[g * 4 + l // 4, pl.ds((l % 4) * 32, 32)],
                    sem,
                )
            return 0

        lax.fori_loop(0, groups, fire, 0)

        def drain(j, _):
            pltpu.make_async_copy(
                table_hbm.at[0], rows_v.at[0, pl.ds(0, 32)], sem
            ).wait()
            return 0

        lax.fori_loop(0, slots, drain, 0)

        def comp(g, _):
            lanes = lax.iota(jnp.int32, 16)
            onehots = [lanes == jnp.int32(l) for l in range(16)]
            dot = jnp.zeros((16,), jnp.float32)
            s1 = jnp.zeros((16,), jnp.float32)
            s2 = jnp.zeros((16,), jnp.float32)
            for l in range(16):
                row = g * 8 + l // 2
                col = (l % 2) * 64
                a0 = rows_v[row, pl.ds(col, 16)]
                a1 = rows_v[row, pl.ds(col + 16, 16)]
                b0 = rows_v[row, pl.ds(col + 32, 16)]
                b1 = rows_v[row, pl.ds(col + 48, 16)]
                dot = jnp.where(onehots[l], jnp.sum(a0 * b0 + a1 * b1), dot)
                s1 = jnp.where(onehots[l], jnp.sum(a0 * a0 + a1 * a1), s1)
                s2 = jnp.where(onehots[l], jnp.sum(b0 * b0 + b1 * b1), s2)
            eps2 = jnp.float32(1e-16)
            t = jnp.maximum(s1, eps2) * jnp.maximum(s2, eps2)
            sim = dot * _rsqrt(t)
            out_v[pl.ds(g * 16, 16)] = jnp.float32(0.5) + jnp.float32(0.5) * sim
            return 0

        lax.fori_loop(0, per_w // 16, comp, 0)
        base = pl.multiple_of(wid * per_w, 8)
        pltpu.sync_copy(out_v, out_hbm.at[pl.ds(base, per_w)])

    return k(table, idx)


def kernel(x, table):
    x = x.reshape(-1, 2)
    batch = x.shape[0]
    slots = (2 * batch) // NW
    idx = x.astype(jnp.int32).reshape(NW, slots // 128, 128)
    return _fused_sc(table, idx, batch)
